# baseline (speedup 1.0000x reference)
import functools

import jax
import jax.numpy as jnp
from jax import lax
from jax.experimental import pallas as pl
from jax.experimental.pallas import tpu as pltpu
from jax.experimental.pallas import tpu_sc as plsc

BATCH = 16384
DIM = 64
NC = 2
NS = 16
NW = NC * NS
BPW = BATCH // NW
CHUNK = 128
NCHUNK = BPW // CHUNK
L = 16
G = CHUNK // L

_mesh = plsc.VectorSubcoreMesh(core_axis_name="c", subcore_axis_name="s")


@functools.partial(
    pl.kernel,
    mesh=_mesh,
    out_type=jax.ShapeDtypeStruct((BATCH,), jnp.float32),
    compiler_params=pltpu.CompilerParams(needs_layout_passes=False),
    scratch_types=[
        pltpu.VMEM((BPW,), jnp.int32),
        pltpu.VMEM((BPW,), jnp.int32),
        pltpu.VMEM((BPW,), jnp.int32),
        pltpu.VMEM((NCHUNK, CHUNK), jnp.int32),
        pltpu.VMEM((NCHUNK, CHUNK), jnp.int32),
        pltpu.VMEM((NCHUNK, CHUNK), jnp.int32),
        pltpu.VMEM((CHUNK, 2 * DIM), jnp.float32),
        pltpu.VMEM((CHUNK, 2 * DIM), jnp.float32),
        pltpu.VMEM((CHUNK, 2 * DIM), jnp.float32),
        pltpu.VMEM((CHUNK, 2 * DIM), jnp.float32),
        pltpu.VMEM((CHUNK, 2 * DIM), jnp.float32),
        pltpu.VMEM((CHUNK, 2 * DIM), jnp.float32),
        pltpu.VMEM((BPW,), jnp.float32),
        pltpu.SemaphoreType.DMA,
    ],
)
def _complex_sc(hs_hbm, rs_hbm, ts_hbm, ent_re_hbm, ent_im_hbm,
                rel_re_hbm, rel_im_hbm, out_hbm,
                hraw, traw, rraw, h2, t2, r2,
                reh, imh, ret, imt, rre, rim, out_v, sem):
    wid = lax.axis_index("s") * NC + lax.axis_index("c")
    base = wid * BPW
    pltpu.sync_copy(hs_hbm.at[pl.ds(base, BPW)], hraw)
    pltpu.sync_copy(ts_hbm.at[pl.ds(base, BPW)], traw)
    pltpu.sync_copy(rs_hbm.at[pl.ds(base, BPW)], rraw)

    def prep(k, _):
        c = k // (CHUNK // L)
        off = (k % (CHUNK // L)) * L
        sl = pl.ds(k * L, L)
        h2[c, pl.ds(off, L)] = hraw[sl] >> 1
        t2[c, pl.ds(off, L)] = traw[sl] >> 1
        r2[c, pl.ds(off, L)] = rraw[sl] >> 1
        return 0

    lax.fori_loop(0, BPW // L, prep, 0)

    def chunk(c, _):
        copies = [
            pltpu.async_copy(ent_re_hbm.at[h2.at[c]], reh, sem),
            pltpu.async_copy(ent_im_hbm.at[h2.at[c]], imh, sem),
            pltpu.async_copy(ent_re_hbm.at[t2.at[c]], ret, sem),
            pltpu.async_copy(ent_im_hbm.at[t2.at[c]], imt, sem),
            pltpu.async_copy(rel_re_hbm.at[r2.at[c]], rre, sem),
            pltpu.async_copy(rel_im_hbm.at[r2.at[c]], rim, sem),
        ]
        for cp in copies:
            cp.wait()

        def group(g, _):
            gbase = c * CHUNK + g * L
            rows = lax.iota(jnp.int32, L) + g * L
            sl = pl.ds(gbase, L)
            hoff = (hraw[sl] & 1) * DIM
            toff = (traw[sl] & 1) * DIM
            roff = (rraw[sl] & 1) * DIM
            acc = jnp.zeros((L,), jnp.float32)
            for j in range(DIM):
                a = plsc.load_gather(reh, [rows, hoff + j])
                b = plsc.load_gather(imh, [rows, hoff + j])
                u = plsc.load_gather(ret, [rows, toff + j])
                v = plsc.load_gather(imt, [rows, toff + j])
                p = plsc.load_gather(rre, [rows, roff + j])
                q = plsc.load_gather(rim, [rows, roff + j])
                acc = acc + p * (a * u + b * v) + q * (a * v - b * u)
            out_v[sl] = acc
            return 0

        lax.fori_loop(0, G, group, 0)
        return 0

    lax.fori_loop(0, NCHUNK, chunk, 0)
    pltpu.sync_copy(out_v, out_hbm.at[pl.ds(base, BPW)])


def kernel(hs, rs, ts, ent_re, ent_im, rel_re, rel_im):
    ent_re2 = ent_re.reshape(-1, 2 * DIM)
    ent_im2 = ent_im.reshape(-1, 2 * DIM)
    rel_re2 = rel_re.reshape(-1, 2 * DIM)
    rel_im2 = rel_im.reshape(-1, 2 * DIM)
    return _complex_sc(hs, rs, ts, ent_re2, ent_im2, rel_re2, rel_im2)


# speedup vs baseline: 1.2944x; 1.2944x over previous
import functools

import jax
import jax.numpy as jnp
from jax import lax
from jax.experimental import pallas as pl
from jax.experimental.pallas import tpu as pltpu
from jax.experimental.pallas import tpu_sc as plsc

BATCH = 16384
DIM = 64
NC = 2
NS = 16
NW = NC * NS
BPW = BATCH // NW
CHUNK = 128
NCHUNK = BPW // CHUNK
L = 16
G = CHUNK // L

_mesh = plsc.VectorSubcoreMesh(core_axis_name="c", subcore_axis_name="s")

_GATHER_DNUMS = lax.GatherDimensionNumbers(
    offset_dims=(), collapsed_slice_dims=(0,), start_index_map=(0,))


def _permute(x, idx):
    return lax.gather(x, idx[:, None], _GATHER_DNUMS, slice_sizes=(1,),
                      mode=lax.GatherScatterMode.PROMISE_IN_BOUNDS)


def _allsum(x, lane):
    for m in (8, 4, 2, 1):
        x = x + _permute(x, lane ^ m)
    return x


@functools.partial(
    pl.kernel,
    mesh=_mesh,
    out_type=jax.ShapeDtypeStruct((BATCH,), jnp.float32),
    compiler_params=pltpu.CompilerParams(needs_layout_passes=False),
    scratch_types=[
        pltpu.VMEM((NCHUNK, CHUNK), jnp.int32),
        pltpu.VMEM((NCHUNK, CHUNK), jnp.int32),
        pltpu.VMEM((NCHUNK, CHUNK), jnp.int32),
        pltpu.VMEM((CHUNK, 2 * DIM), jnp.float32),
        pltpu.VMEM((CHUNK, 2 * DIM), jnp.float32),
        pltpu.VMEM((CHUNK, 2 * DIM), jnp.float32),
        pltpu.VMEM((BPW,), jnp.float32),
        pltpu.SemaphoreType.DMA,
    ],
)
def _complex_sc(hs_hbm, rs_hbm, ts_hbm, ent_hbm, rel_hbm, out_hbm,
                h2, t2, r2, ch, ct, cr, out_v, sem):
    wid = lax.axis_index("s") * NC + lax.axis_index("c")
    base = wid * BPW
    for c in range(NCHUNK):
        off = base + c * CHUNK
        pltpu.sync_copy(hs_hbm.at[pl.ds(off, CHUNK)], h2.at[c])
        pltpu.sync_copy(ts_hbm.at[pl.ds(off, CHUNK)], t2.at[c])
        pltpu.sync_copy(rs_hbm.at[pl.ds(off, CHUNK)], r2.at[c])

    def chunk(c, _):
        copies = [
            pltpu.async_copy(ent_hbm.at[h2.at[c]], ch, sem),
            pltpu.async_copy(ent_hbm.at[t2.at[c]], ct, sem),
            pltpu.async_copy(rel_hbm.at[r2.at[c]], cr, sem),
        ]
        for cp in copies:
            cp.wait()

        def group(g, _):
            lane = lax.iota(jnp.int32, L)
            scores = jnp.zeros((L,), jnp.float32)
            for k in range(L):
                i = g * L + k
                acc = jnp.zeros((L,), jnp.float32)
                for j in range(DIM // L):
                    re_sl = pl.ds(j * L, L)
                    im_sl = pl.ds(DIM + j * L, L)
                    a = ch[i, re_sl]
                    b = ch[i, im_sl]
                    u = ct[i, re_sl]
                    v = ct[i, im_sl]
                    p = cr[i, re_sl]
                    q = cr[i, im_sl]
                    acc = acc + p * (a * u + b * v) + q * (a * v - b * u)
                scores = jnp.where(lane == k, _allsum(acc, lane), scores)
            out_v[pl.ds(c * CHUNK + g * L, L)] = scores
            return 0

        lax.fori_loop(0, G, group, 0)
        return 0

    lax.fori_loop(0, NCHUNK, chunk, 0)
    pltpu.sync_copy(out_v, out_hbm.at[pl.ds(base, BPW)])


def kernel(hs, rs, ts, ent_re, ent_im, rel_re, rel_im):
    ent_cat = jnp.concatenate([ent_re, ent_im], axis=1)
    rel_cat = jnp.concatenate([rel_re, rel_im], axis=1)
    return _complex_sc(hs, rs, ts, ent_cat, rel_cat)
